# baseline (speedup 1.0000x reference)
import functools

import jax
import jax.numpy as jnp
from jax import lax
from jax.experimental import pallas as pl
from jax.experimental.pallas import tpu as pltpu
from jax.experimental.pallas import tpu_sc as plsc

BATCH = 16384
FACTOR = 32
TABLE = 1000000
NC = 2
NS = 16
NW = NC * NS
B_PER_W = BATCH // NW
CHUNK = 128
NCHUNK = B_PER_W // CHUNK
WIN = 12
WS_MAX = (1000064 - WIN * CHUNK) // CHUNK


def _gather_body(sidx_hbm, tab_hbm, emb_hbm, sidx_v, win_v, emb_v):
    wid = lax.axis_index("s") * NC + lax.axis_index("c")

    pltpu.sync_copy(sidx_hbm.at[pl.ds(wid * NCHUNK, NCHUNK), :], sidx_v)

    iota = lax.iota(jnp.int32, 16)

    def group_body(g, ws):
        j = lax.shift_right_logical(g, 3)
        cb = lax.shift_left(lax.bitwise_and(g, 7), 4)
        sv = sidx_v[j, pl.ds(cb, 16)]
        for l in range(16):
            r = sv[l]
            blk = lax.shift_right_logical(r, 7)
            refresh = blk >= ws + WIN
            ws = jnp.where(refresh, jnp.minimum(blk, WS_MAX), ws)

            @pl.when(refresh)
            def _(ws=ws):
                start = pl.multiple_of(lax.shift_left(ws, 7), 128)
                pltpu.sync_copy(tab_hbm.at[:, pl.ds(start, WIN * CHUNK)],
                                win_v)

            col = jnp.broadcast_to(r - lax.shift_left(ws, 7), (16,))
            lo = plsc.load_gather(win_v, [iota, col])
            hi = plsc.load_gather(win_v, [iota + 16, col])
            off = lax.shift_left(g, 9) + 32 * l
            emb_v[pl.ds(off, 16)] = lo
            emb_v[pl.ds(off + 16, 16)] = hi
        return ws

    lax.fori_loop(0, B_PER_W // 16, group_body, jnp.int32(-2 * WIN))

    pltpu.sync_copy(emb_v, emb_hbm.at[pl.ds(wid * B_PER_W * FACTOR,
                                            B_PER_W * FACTOR)])


def _pair_body(ue_hbm, ie_hbm, mix_hbm, pu_hbm, out_hbm,
               mix_v, pu_v, ue_v, ir_v, pr_v, sem):
    wid = lax.axis_index("s") * NC + lax.axis_index("c")

    pltpu.sync_copy(mix_hbm.at[pl.ds(wid * NCHUNK, NCHUNK), :], mix_v)
    pltpu.sync_copy(pu_hbm.at[pl.ds(wid * NCHUNK, NCHUNK), :], pu_v)
    pltpu.sync_copy(ue_hbm.at[pl.ds(wid * B_PER_W * FACTOR,
                                    B_PER_W * FACTOR)], ue_v)

    for j in range(NCHUNK):
        pltpu.async_copy(ie_hbm.at[mix_v.at[j]], ir_v, sem).wait()

        def row_body(k, _, j=j):
            off = lax.shift_left(j, 12) + lax.shift_left(k, 5)
            for c in (0, 16):
                pr_v[k, pl.ds(c, 16)] = (
                    ue_v[pl.ds(off + c, 16)] * ir_v[k, pl.ds(c, 16)])
            return ()

        lax.fori_loop(0, CHUNK, row_body, ())
        pltpu.async_copy(pr_v, out_hbm.at[pu_v.at[j]], sem).wait()


@jax.jit
def _gmf(su2, si2, mix2, pu2, utabT, itabT):
    mesh = plsc.VectorSubcoreMesh(core_axis_name="c", subcore_axis_name="s")
    gather_k = functools.partial(
        pl.kernel,
        mesh=mesh,
        compiler_params=pltpu.CompilerParams(needs_layout_passes=False),
        out_type=jax.ShapeDtypeStruct((BATCH * FACTOR,), jnp.float32),
        scratch_types=[
            pltpu.VMEM((NCHUNK, CHUNK), jnp.int32),
            pltpu.VMEM((FACTOR, WIN * CHUNK), jnp.float32),
            pltpu.VMEM((B_PER_W * FACTOR,), jnp.float32),
        ],
    )(_gather_body)
    ue1 = gather_k(su2, utabT)
    ie1 = gather_k(si2, itabT)
    ie2 = ie1.reshape(BATCH, FACTOR)
    pair_k = functools.partial(
        pl.kernel,
        mesh=mesh,
        compiler_params=pltpu.CompilerParams(
            needs_layout_passes=False, use_tc_tiling_on_sc=False),
        out_type=jax.ShapeDtypeStruct((BATCH, FACTOR), jnp.float32),
        scratch_types=[
            pltpu.VMEM((NCHUNK, CHUNK), jnp.int32),
            pltpu.VMEM((NCHUNK, CHUNK), jnp.int32),
            pltpu.VMEM((B_PER_W * FACTOR,), jnp.float32),
            pltpu.VMEM((CHUNK, FACTOR), jnp.float32),
            pltpu.VMEM((CHUNK, FACTOR), jnp.float32),
            pltpu.SemaphoreType.DMA,
        ],
    )(_pair_body)
    return pair_k(ue1, ie2, mix2, pu2)


def kernel(user_indices, item_indices, user_table, item_table):
    iota = lax.iota(jnp.int32, BATCH)
    su, pu = lax.sort_key_val(user_indices.astype(jnp.int32), iota)
    si, pi = lax.sort_key_val(item_indices.astype(jnp.int32), iota)
    inv_i = jnp.zeros((BATCH,), jnp.int32).at[pi].set(iota)
    mix = inv_i[pu]
    out = _gmf(su.reshape(BATCH // CHUNK, CHUNK),
               si.reshape(BATCH // CHUNK, CHUNK),
               mix.reshape(BATCH // CHUNK, CHUNK),
               pu.reshape(BATCH // CHUNK, CHUNK),
               user_table.T, item_table.T)
    return out


# speedup vs baseline: 1.1156x; 1.1156x over previous
import functools

import jax
import jax.numpy as jnp
from jax import lax
from jax.experimental import pallas as pl
from jax.experimental.pallas import tpu as pltpu
from jax.experimental.pallas import tpu_sc as plsc

BATCH = 16384
FACTOR = 32
TABLE = 1000000
NC = 2
NS = 16
NW = NC * NS
B_PER_W = BATCH // NW
CHUNK = 128
NCHUNK = B_PER_W // CHUNK
NRING = 8


def _gmf_body(uidx_hbm, iidx_hbm, utab_hbm, itab_hbm, out_hbm,
              uidx_v, iidx_v, uring, iring, out_v, usem, isem):
    wid = lax.axis_index("s") * NC + lax.axis_index("c")
    base = wid * B_PER_W

    pltpu.sync_copy(uidx_hbm.at[pl.ds(wid * NCHUNK, NCHUNK), :], uidx_v)
    pltpu.sync_copy(iidx_hbm.at[pl.ds(wid * NCHUNK, NCHUNK), :], iidx_v)

    iota = lax.iota(jnp.int32, 16)

    def fetch(r, ring, tab, sem, slot):
        blk = pl.multiple_of(lax.bitwise_and(r, ~127), 128)
        pltpu.async_copy(tab.at[:, pl.ds(blk, CHUNK)],
                         ring.at[slot], sem.at[slot])

    def drain(ring, tab, sem, slot):
        pltpu.make_async_copy(tab.at[:, pl.ds(0, CHUNK)],
                              ring.at[slot], sem.at[slot]).wait()

    def extract(ring, slot, rmod):
        cols = jnp.broadcast_to(rmod, (16,))
        lo = plsc.load_gather(ring.at[slot], [iota, cols])
        hi = plsc.load_gather(ring.at[slot], [iota + 16, cols])
        return lo, hi

    def group_body(g, _):
        j = lax.shift_right_logical(g, 3)
        cb = lax.shift_left(lax.bitwise_and(g, 7), 4)
        uv = uidx_v[j, pl.ds(cb, 16)]
        iv = iidx_v[j, pl.ds(cb, 16)]
        b0 = lax.shift_left(g, 4)

        for l in range(NRING):
            fetch(uv[l], uring, utab_hbm, usem, l)
            fetch(iv[l], iring, itab_hbm, isem, l)
        for l in range(16):
            slot = l % NRING
            drain(uring, utab_hbm, usem, slot)
            drain(iring, itab_hbm, isem, slot)
            um = lax.bitwise_and(uv[l], 127)
            im = lax.bitwise_and(iv[l], 127)
            ulo, uhi = extract(uring, slot, um)
            ilo, ihi = extract(iring, slot, im)
            bcol = jnp.broadcast_to(b0 + l, (16,))
            plsc.store_scatter(out_v, [iota, bcol], ulo * ilo)
            plsc.store_scatter(out_v, [iota + 16, bcol], uhi * ihi)
            if l + NRING < 16:
                fetch(uv[l + NRING], uring, utab_hbm, usem, slot)
                fetch(iv[l + NRING], iring, itab_hbm, isem, slot)
        return ()

    lax.fori_loop(0, B_PER_W // 16, group_body, ())

    pltpu.sync_copy(out_v, out_hbm.at[:, pl.ds(base, B_PER_W)])


@jax.jit
def _gmf(uidx2, iidx2, utabT, itabT):
    mesh = plsc.VectorSubcoreMesh(core_axis_name="c", subcore_axis_name="s")
    kfn = functools.partial(
        pl.kernel,
        mesh=mesh,
        compiler_params=pltpu.CompilerParams(needs_layout_passes=False),
        out_type=jax.ShapeDtypeStruct((FACTOR, BATCH), jnp.float32),
        scratch_types=[
            pltpu.VMEM((NCHUNK, CHUNK), jnp.int32),
            pltpu.VMEM((NCHUNK, CHUNK), jnp.int32),
            pltpu.VMEM((NRING, FACTOR, CHUNK), jnp.float32),
            pltpu.VMEM((NRING, FACTOR, CHUNK), jnp.float32),
            pltpu.VMEM((FACTOR, B_PER_W), jnp.float32),
            pltpu.SemaphoreType.DMA((NRING,)),
            pltpu.SemaphoreType.DMA((NRING,)),
        ],
    )(_gmf_body)
    return kfn(uidx2, iidx2, utabT, itabT)


def kernel(user_indices, item_indices, user_table, item_table):
    uidx2 = user_indices.astype(jnp.int32).reshape(BATCH // CHUNK, CHUNK)
    iidx2 = item_indices.astype(jnp.int32).reshape(BATCH // CHUNK, CHUNK)
    outT = _gmf(uidx2, iidx2, user_table.T, item_table.T)
    return outT.T
